# baseline (speedup 1.0000x reference)
import jax
import jax.numpy as jnp
from jax.experimental import pallas as pl
from jax.experimental.pallas import tpu as pltpu

NSTREAMS = 2
BLOCK_T = 512


def _router_kernel(*refs):
    x_refs = refs[:NSTREAMS]
    w_ref = refs[NSTREAMS]
    o_ref = refs[NSTREAMS + 1]
    w = w_ref[...]
    for s, x_ref in enumerate(x_refs):
        x = x_ref[...]
        logits = jnp.dot(x, w, preferred_element_type=jnp.float32)
        m = jnp.max(logits, axis=-1, keepdims=True)
        e = jnp.exp(logits - m)
        o_ref[s] = e / jnp.sum(e, axis=-1, keepdims=True)


def kernel(states, W):
    T, D = states.shape
    E = W.shape[0]
    wt = W.T
    steps = T // (NSTREAMS * BLOCK_T)

    def x_index(stream):
        return lambda i: (stream * steps + i, 0)

    out = pl.pallas_call(
        _router_kernel,
        grid=(steps,),
        in_specs=[pl.BlockSpec((BLOCK_T, D), x_index(s)) for s in range(NSTREAMS)]
        + [pl.BlockSpec((D, E), lambda i: (0, 0))],
        out_specs=pl.BlockSpec((NSTREAMS, BLOCK_T, E), lambda i: (0, i, 0)),
        out_shape=jax.ShapeDtypeStruct((NSTREAMS, T // NSTREAMS, E), jnp.float32),
        compiler_params=pltpu.CompilerParams(
            vmem_limit_bytes=100 * 1024 * 1024,
        ),
    )(*([states] * NSTREAMS), wt)
    return out.reshape(T, E)


# speedup vs baseline: 1.0697x; 1.0697x over previous
import jax
import jax.numpy as jnp
from jax.experimental import pallas as pl
from jax.experimental.pallas import tpu as pltpu

BLOCK_T = 1024


def _router_kernel(x_ref, w_ref, o_ref):
    x = x_ref[...]
    w = w_ref[...]
    logits = jnp.dot(x, w, preferred_element_type=jnp.float32)
    m = jnp.max(logits, axis=-1, keepdims=True)
    e = jnp.exp(logits - m)
    o_ref[...] = e / jnp.sum(e, axis=-1, keepdims=True)


def kernel(states, W):
    T, D = states.shape
    E = W.shape[0]
    wt = W.T
    return pl.pallas_call(
        _router_kernel,
        grid=(T // BLOCK_T,),
        in_specs=[
            pl.BlockSpec((BLOCK_T, D), lambda i: (i, 0)),
            pl.BlockSpec((D, E), lambda i: (0, 0)),
        ],
        out_specs=pl.BlockSpec((BLOCK_T, E), lambda i: (i, 0)),
        out_shape=jax.ShapeDtypeStruct((T, E), jnp.float32),
        compiler_params=pltpu.CompilerParams(
            dimension_semantics=("parallel",),
            vmem_limit_bytes=100 * 1024 * 1024,
        ),
    )(states, wt)
